# Initial kernel scaffold; baseline (speedup 1.0000x reference)
import jax
import jax.numpy as jnp
from jax.experimental import pallas as pl


def kernel(images):
    raise NotImplementedError("write your pallas kernel here")


# speedup vs baseline: 30.2597x; 30.2597x over previous
import jax
import jax.numpy as jnp
from jax import lax
from jax.experimental import pallas as pl
from jax.experimental.pallas import tpu as pltpu

_RING_SLICES = ((0, 18), (22, 26), (30, 34), (38, 42), (46, 64))


def _fov_kernel(img_ref, out_ref):
    r = lax.broadcasted_iota(jnp.int32, (128, 512), 0)
    c = lax.broadcasted_iota(jnp.int32, (128, 512), 1)
    a4 = (c // 4 == r).astype(jnp.float32)
    a2 = (c[:, :256] // 2 == r[:, :256]).astype(jnp.float32)

    for ch in range(3):
        img = img_ref[ch]
        p0 = jnp.floor(img[192:320, 192:320])
        sub = img[128:384, 128:384]
        t1 = lax.dot_general(a2, sub, (((1,), (0,)), ((), ())),
                             precision=lax.Precision.HIGHEST,
                             preferred_element_type=jnp.float32)
        p1 = jnp.floor(
            lax.dot_general(t1, a2, (((1,), (1,)), ((), ())),
                            precision=lax.Precision.HIGHEST,
                            preferred_element_type=jnp.float32) * 0.25)
        t2 = lax.dot_general(a4, img, (((1,), (0,)), ((), ())),
                             precision=lax.Precision.HIGHEST,
                             preferred_element_type=jnp.float32)
        p2 = jnp.floor(
            lax.dot_general(t2, a4, (((1,), (1,)), ((), ())),
                            precision=lax.Precision.HIGHEST,
                            preferred_element_type=jnp.float32) * 0.0625)

        for base, plane, full in ((0, p0, True), (64, p1, False), (112, p2, False)):
            tiles = plane.reshape(8, 16, 8, 16).transpose(0, 2, 1, 3)
            tiles = tiles.reshape(64, 16, 16)
            if full:
                out_ref[0:64, ch] = tiles
            else:
                off = base
                for s0, s1 in _RING_SLICES:
                    out_ref[off:off + (s1 - s0), ch] = tiles[s0:s1]
                    off += s1 - s0


def kernel(images):
    return pl.pallas_call(
        _fov_kernel,
        out_shape=jax.ShapeDtypeStruct((160, 3, 16, 16), jnp.float32),
    )(images)
